# baseline (speedup 1.0000x reference)
import jax
import jax.numpy as jnp
from jax import lax
from jax.experimental import pallas as pl
from jax.experimental.pallas import tpu as pltpu
from jax.experimental.pallas import tpu_sc as plsc

Q = 4096
D = 64
KS = 16
ROW = KS + 1
NTR = 1000000
NC, NS, L = 2, 16, 16
NW = NC * NS
QW = Q // NW
CQ = 8
NCH = QW // CQ
NSL = Q + 1
NSL_PAD = ((NSL + L - 1) // L) * L
CB = 8192


def _tp_body(in_ref, out_ref):
    out_ref[...] = in_ref[...].T


def _transpose_table(tt):
    grid = (NTR + CB - 1) // CB
    return pl.pallas_call(
        _tp_body,
        grid=(grid,),
        in_specs=[pl.BlockSpec((D, CB), lambda i: (0, i))],
        out_specs=pl.BlockSpec((CB, D), lambda i: (i, 0)),
        out_shape=jax.ShapeDtypeStruct((NTR, D), jnp.float32),
    )(tt)


def _body(qf, ids, table, kvec, out, slices,
          qblk_v, idx_v, comb_v, slc_v, kv_v, sem):
    wid = lax.axis_index("s") * NC + lax.axis_index("c")
    q0w = wid * QW

    @pl.when(wid == 0)
    def _():
        pltpu.sync_copy(kvec, kv_v)
        step = kv_v[...] + 1

        def sbody(i, carry):
            off = pl.multiple_of(i * L, 8)
            slc_v[pl.ds(off, L)] = (lax.iota(jnp.int32, L) + i * L) * step
            return carry

        lax.fori_loop(0, NSL_PAD // L, sbody, 0)
        pltpu.sync_copy(slc_v.at[pl.ds(0, NSL)], slices)

    pltpu.sync_copy(qf.at[pl.ds(q0w, QW)], qblk_v)

    def chunk(c, carry):
        q0 = q0w + c * CQ
        i0 = pl.multiple_of(q0 * KS, 8)
        pltpu.sync_copy(ids.at[pl.ds(i0, CQ * KS)], idx_v)
        cps = []
        for j in range(CQ):
            idxj = idx_v[pl.ds(j * KS, KS)]
            cps.append(pltpu.async_copy(
                table.at[idxj], comb_v.at[pl.ds(j * ROW + 1, KS)], sem))
        for j in range(CQ):
            r = c * CQ + j
            for t in range(D // L):
                comb_v[j * ROW, pl.ds(t * L, L)] = qblk_v[r, pl.ds(t * L, L)]
        for cp in cps:
            cp.wait()
        pltpu.sync_copy(comb_v, out.at[pl.ds(q0 * ROW, CQ * ROW)])
        return carry

    lax.fori_loop(0, NCH, chunk, 0)


@jax.jit
def _nn_gather(query_feats, ids_flat, train_table_t, kvec):
    table_rm = _transpose_table(train_table_t)
    mesh = plsc.VectorSubcoreMesh(core_axis_name="c", subcore_axis_name="s")
    call = pl.kernel(
        _body,
        out_type=[
            jax.ShapeDtypeStruct((Q * ROW, D), jnp.float32),
            jax.ShapeDtypeStruct((NSL,), jnp.int32),
        ],
        mesh=mesh,
        scratch_types=[
            pltpu.VMEM((QW, D), jnp.float32),
            pltpu.VMEM((CQ * KS,), jnp.int32),
            pltpu.VMEM((CQ * ROW, D), jnp.float32),
            pltpu.VMEM((NSL_PAD,), jnp.int32),
            pltpu.VMEM((L,), jnp.int32),
            pltpu.SemaphoreType.DMA,
        ],
        compiler_params=pltpu.CompilerParams(use_tc_tiling_on_sc=False),
    )
    return call(query_feats, ids_flat, table_rm, kvec)


def kernel(query_feats, knn_ids, train_table, k):
    ids_flat = knn_ids.reshape(-1).astype(jnp.int32)
    kvec = jnp.full((L,), k, dtype=jnp.int32)
    neighbor_list, neighbor_slices = _nn_gather(
        query_feats, ids_flat, train_table.T, kvec)
    return neighbor_list, neighbor_slices


# speedup vs baseline: 1.9738x; 1.9738x over previous
import jax
import jax.numpy as jnp
from jax import lax
from jax.experimental import pallas as pl
from jax.experimental.pallas import tpu as pltpu
from jax.experimental.pallas import tpu_sc as plsc

Q = 4096
D = 64
DP = 128
KS = 16
ROW = KS + 1
NTR = 1000000
NC, NS, L = 2, 16, 16
NW = NC * NS
QW = Q // NW
CQ = 8
NCH = QW // CQ
NSL = Q + 1
NSL_PAD = ((NSL + L - 1) // L) * L
CB = 8192


def _tp_body(in_ref, out_ref):
    out_ref[:, 0:D] = in_ref[...].T


def _transpose_table(tt):
    grid = (NTR + CB - 1) // CB
    return pl.pallas_call(
        _tp_body,
        grid=(grid,),
        in_specs=[pl.BlockSpec((D, CB), lambda i: (0, i))],
        out_specs=pl.BlockSpec((CB, DP), lambda i: (i, 0)),
        out_shape=jax.ShapeDtypeStruct((NTR, DP), jnp.float32),
    )(tt)


def _body(qf, ids, table, kvec, out, slices,
          qblk_v, idx_v, comb_v, slc_v, kv_v, sem):
    wid = lax.axis_index("s") * NC + lax.axis_index("c")
    q0w = wid * QW

    @pl.when(wid == 0)
    def _():
        pltpu.sync_copy(kvec, kv_v)
        step = kv_v[...] + 1

        def sbody(i, carry):
            off = pl.multiple_of(i * L, 8)
            slc_v[pl.ds(off, L)] = (lax.iota(jnp.int32, L) + i * L) * step
            return carry

        lax.fori_loop(0, NSL_PAD // L, sbody, 0)
        pltpu.sync_copy(slc_v.at[pl.ds(0, NSL)], slices)

    pltpu.sync_copy(qf.at[pl.ds(q0w, QW)], qblk_v)

    def chunk(c, carry):
        q0 = q0w + c * CQ
        i0 = pl.multiple_of(q0 * KS, 8)
        pltpu.sync_copy(ids.at[pl.ds(i0, CQ * KS)], idx_v)
        cps = []
        for j in range(CQ):
            idxj = idx_v[pl.ds(j * KS, KS)]
            cps.append(pltpu.async_copy(
                table.at[idxj], comb_v.at[pl.ds(j * ROW + 1, KS)], sem))
        for j in range(CQ):
            r = c * CQ + j
            for t in range(D // L):
                comb_v[j * ROW, pl.ds(t * L, L)] = qblk_v[r, pl.ds(t * L, L)]
        for cp in cps:
            cp.wait()
        pltpu.sync_copy(comb_v.at[:, pl.ds(0, D)],
                        out.at[pl.ds(q0 * ROW, CQ * ROW)])
        return carry

    lax.fori_loop(0, NCH, chunk, 0)


@jax.jit
def _nn_gather(query_feats, ids_flat, train_table_t, kvec):
    table_rm = _transpose_table(train_table_t)
    table_lin = table_rm.reshape(NTR * DP)
    table_lin = table_lin.reshape(NTR, DP)
    mesh = plsc.VectorSubcoreMesh(core_axis_name="c", subcore_axis_name="s")
    call = pl.kernel(
        _body,
        out_type=[
            jax.ShapeDtypeStruct((Q * ROW, D), jnp.float32),
            jax.ShapeDtypeStruct((NSL,), jnp.int32),
        ],
        mesh=mesh,
        scratch_types=[
            pltpu.VMEM((QW, D), jnp.float32),
            pltpu.VMEM((CQ * KS,), jnp.int32),
            pltpu.VMEM((CQ * ROW, DP), jnp.float32),
            pltpu.VMEM((NSL_PAD,), jnp.int32),
            pltpu.VMEM((L,), jnp.int32),
            pltpu.SemaphoreType.DMA,
        ],
        compiler_params=pltpu.CompilerParams(use_tc_tiling_on_sc=False),
    )
    return call(query_feats, ids_flat, table_lin, kvec)


def kernel(query_feats, knn_ids, train_table, k):
    ids_flat = knn_ids.reshape(-1).astype(jnp.int32)
    kvec = jnp.full((L,), k, dtype=jnp.int32)
    neighbor_list, neighbor_slices = _nn_gather(
        query_feats, ids_flat, train_table.T, kvec)
    return neighbor_list, neighbor_slices


# speedup vs baseline: 2.1626x; 1.0957x over previous
import jax
import jax.numpy as jnp
from jax import lax
from jax.experimental import pallas as pl
from jax.experimental.pallas import tpu as pltpu
from jax.experimental.pallas import tpu_sc as plsc

Q = 4096
D = 64
DP = 128
KS = 16
ROW = KS + 1
NTR = 1000000
NC, NS, L = 2, 16, 16
NW = NC * NS
QW = Q // NW
CQ = 8
NCH = QW // CQ
NSL = Q + 1
NSL_PAD = ((NSL + L - 1) // L) * L
CB = 16384


def _tp_body(in_ref, out_ref):
    out_ref[:, 0:D] = in_ref[...].T


def _transpose_table(tt):
    grid = (NTR + CB - 1) // CB
    return pl.pallas_call(
        _tp_body,
        grid=(grid,),
        in_specs=[pl.BlockSpec((D, CB), lambda i: (0, i))],
        out_specs=pl.BlockSpec((CB, DP), lambda i: (i, 0)),
        out_shape=jax.ShapeDtypeStruct((NTR, DP), jnp.float32),
    )(tt)


def _body(qf, ids, table, kvec, out, slices,
          qblk_v, idx_v, comb_v, slc_v, kv_v, sem):
    wid = lax.axis_index("s") * NC + lax.axis_index("c")
    q0w = wid * QW

    @pl.when(wid == 0)
    def _():
        pltpu.sync_copy(kvec, kv_v)
        step = kv_v[...] + 1

        def sbody(i, carry):
            off = pl.multiple_of(i * L, 8)
            slc_v[pl.ds(off, L)] = (lax.iota(jnp.int32, L) + i * L) * step
            return carry

        lax.fori_loop(0, NSL_PAD // L, sbody, 0)
        pltpu.sync_copy(slc_v.at[pl.ds(0, NSL)], slices)

    pltpu.sync_copy(qf.at[pl.ds(q0w, QW)], qblk_v)

    def chunk(c, carry):
        q0 = q0w + c * CQ
        i0 = pl.multiple_of(q0 * KS, 8)
        pltpu.sync_copy(ids.at[pl.ds(i0, CQ * KS)], idx_v)
        cps = []
        for j in range(CQ):
            idxj = idx_v[pl.ds(j * KS, KS)] * 2
            cps.append(pltpu.async_copy(
                table.at[idxj], comb_v.at[pl.ds(j * ROW + 1, KS)], sem))
        for j in range(CQ):
            r = c * CQ + j
            for t in range(D // L):
                comb_v[j * ROW, pl.ds(t * L, L)] = qblk_v[r, pl.ds(t * L, L)]
        for cp in cps:
            cp.wait()
        pltpu.sync_copy(comb_v, out.at[pl.ds(q0 * ROW, CQ * ROW)])
        return carry

    lax.fori_loop(0, NCH, chunk, 0)


@jax.jit
def _nn_gather(query_feats, ids_flat, train_table_t, kvec):
    table_rm = _transpose_table(train_table_t)
    table_lin = table_rm.reshape(2 * NTR, D)
    mesh = plsc.VectorSubcoreMesh(core_axis_name="c", subcore_axis_name="s")
    call = pl.kernel(
        _body,
        out_type=[
            jax.ShapeDtypeStruct((Q * ROW, D), jnp.float32),
            jax.ShapeDtypeStruct((NSL,), jnp.int32),
        ],
        mesh=mesh,
        scratch_types=[
            pltpu.VMEM((QW, D), jnp.float32),
            pltpu.VMEM((CQ * KS,), jnp.int32),
            pltpu.VMEM((CQ * ROW, D), jnp.float32),
            pltpu.VMEM((NSL_PAD,), jnp.int32),
            pltpu.VMEM((L,), jnp.int32),
            pltpu.SemaphoreType.DMA,
        ],
        compiler_params=pltpu.CompilerParams(use_tc_tiling_on_sc=False),
    )
    return call(query_feats, ids_flat, table_lin, kvec)


def kernel(query_feats, knn_ids, train_table, k):
    ids_flat = knn_ids.reshape(-1).astype(jnp.int32)
    kvec = jnp.full((L,), k, dtype=jnp.int32)
    neighbor_list, neighbor_slices = _nn_gather(
        query_feats, ids_flat, train_table.T, kvec)
    return neighbor_list, neighbor_slices
